# Initial kernel scaffold; baseline (speedup 1.0000x reference)
import jax
import jax.numpy as jnp
from jax.experimental import pallas as pl


def kernel(feat, edge_index, W, al, ar):
    raise NotImplementedError("write your pallas kernel here")


# speedup vs baseline: 11.3300x; 11.3300x over previous
import functools

import jax
import jax.numpy as jnp
from jax import lax
from jax.experimental import pallas as pl
from jax.experimental.pallas import tpu as pltpu
from jax.experimental.pallas import tpu_sc as plsc

N_NODES = 10000
D = 128
N_EDGES = 320000

NC = 2
NS = 16
NW = NC * NS
MICRO = 128
NBUF = 2
M_PER_TILE = 80
CH = 16
NCHUNK = M_PER_TILE // CH
EP = NW * M_PER_TILE * MICRO
ROWS2D = EP // MICRO
ACC_ROWS = 10240
PAD_DST = ACC_ROWS - 1
ZROWS = ACC_ROWS // NS


def _mm_body(x_ref, w_ref, o_ref):
    o_ref[...] = jnp.dot(x_ref[...], w_ref[...],
                         preferred_element_type=jnp.float32)


def _project(feat, W):
    return pl.pallas_call(
        _mm_body,
        grid=(10,),
        in_specs=[
            pl.BlockSpec((N_NODES // 10, D), lambda i: (i, 0)),
            pl.BlockSpec((D, D), lambda i: (0, 0)),
        ],
        out_specs=pl.BlockSpec((N_NODES // 10, D), lambda i: (i, 0)),
        out_shape=jax.ShapeDtypeStruct((N_NODES, D), jnp.float32),
    )(feat, W)


def _add_body(p_ref, o_ref):
    o_ref[...] = p_ref[0] + p_ref[1]


def _combine(partials):
    return pl.pallas_call(
        _add_body,
        grid=(10,),
        in_specs=[pl.BlockSpec((2, N_NODES // 10, D), lambda i: (0, i, 0))],
        out_specs=pl.BlockSpec((N_NODES // 10, D), lambda i: (i, 0)),
        out_shape=jax.ShapeDtypeStruct((N_NODES, D), jnp.float32),
    )(partials)


@functools.partial(
    pl.kernel,
    out_type=jax.ShapeDtypeStruct((NC, ACC_ROWS, D), jnp.float32),
    mesh=plsc.VectorSubcoreMesh(core_axis_name="c", subcore_axis_name="s"),
    scratch_types=[
        pltpu.VMEM_SHARED((ACC_ROWS, D), jnp.float32),
        pltpu.VMEM((CH, MICRO), jnp.int32),
        pltpu.VMEM((CH, MICRO), jnp.int32),
        pltpu.VMEM((NBUF, MICRO, D), jnp.float32),
        pltpu.SemaphoreType.DMA,
        pltpu.SemaphoreType.DMA,
    ],
)
def _sc_segment_sum(h_hbm, src_hbm, dst_hbm, out_hbm,
                    acc, srcv, dstv, bufs, s0, s1):
    sems = (s0, s1)
    c = lax.axis_index("c")
    s = lax.axis_index("s")
    wid = c * NS + s
    row0 = wid * M_PER_TILE

    zbuf = bufs.at[0]
    zero16 = jnp.zeros((16,), jnp.float32)

    def _zero_row(i, _):
        for cc in range(D // 16):
            zbuf[i, pl.ds(cc * 16, 16)] = zero16
        return 0

    lax.fori_loop(0, MICRO, _zero_row, 0)
    zbase = s * ZROWS
    for k in range(ZROWS // MICRO):
        pltpu.sync_copy(zbuf, acc.at[pl.ds(zbase + k * MICRO, MICRO)])

    plsc.subcore_barrier()

    def _chunk(k, _):
        pltpu.sync_copy(src_hbm.at[pl.ds(row0 + k * CH, CH)], srcv)
        pltpu.sync_copy(dst_hbm.at[pl.ds(row0 + k * CH, CH)], dstv)

        def _pair(j, _):
            descs = []
            for b in range(NBUF):
                m = j * NBUF + b
                descs.append(pltpu.async_copy(
                    h_hbm.at[srcv.at[m]], bufs.at[b], sems[b]))
            for b in range(NBUF):
                m = j * NBUF + b
                descs[b].wait()
                pltpu.sync_copy(bufs.at[b], acc.at[dstv.at[m]], add=True)
            return 0

        lax.fori_loop(0, CH // NBUF, _pair, 0)
        return 0

    lax.fori_loop(0, NCHUNK, _chunk, 0)

    plsc.subcore_barrier()

    obase = s * ZROWS
    pltpu.sync_copy(acc.at[pl.ds(obase, ZROWS)],
                    out_hbm.at[c, pl.ds(obase, ZROWS)])


def kernel(feat, edge_index, W, al, ar):
    del al, ar
    src = edge_index[0]
    dst = edge_index[1]
    pad = EP - N_EDGES
    src2d = jnp.concatenate(
        [src, jnp.zeros((pad,), jnp.int32)]).reshape(ROWS2D, MICRO)
    dst2d = jnp.concatenate(
        [dst, jnp.full((pad,), PAD_DST, jnp.int32)]).reshape(ROWS2D, MICRO)
    h = _project(feat, W)
    partials = _sc_segment_sum(h, src2d, dst2d)
    return _combine(partials)


# speedup vs baseline: 30.7205x; 2.7114x over previous
import functools

import jax
import jax.numpy as jnp
from jax import lax
from jax.experimental import pallas as pl
from jax.experimental.pallas import tpu as pltpu
from jax.experimental.pallas import tpu_sc as plsc

N_NODES = 10000
D = 128
N_EDGES = 320000

NC = 2
NS = 16
NW = NC * NS
MICRO = 128
NBUF = 2
M_PER_TILE = 80
CH = 16
NCHUNK = M_PER_TILE // CH
EP = NW * M_PER_TILE * MICRO
ROWS2D = EP // MICRO
ACC_ROWS = 10240
PAD_DST = ACC_ROWS - 1
ZROWS = ACC_ROWS // NS


def _mm_body(x_ref, w_ref, o_ref):
    o_ref[...] = jnp.dot(x_ref[...], w_ref[...],
                         preferred_element_type=jnp.float32)


def _project(feat, W):
    return pl.pallas_call(
        _mm_body,
        grid=(10,),
        in_specs=[
            pl.BlockSpec((N_NODES // 10, D), lambda i: (i, 0)),
            pl.BlockSpec((D, D), lambda i: (0, 0)),
        ],
        out_specs=pl.BlockSpec((N_NODES // 10, D), lambda i: (i, 0)),
        out_shape=jax.ShapeDtypeStruct((N_NODES, D), jnp.float32),
    )(feat, W)


def _add_body(p_ref, o_ref):
    o_ref[...] = p_ref[0] + p_ref[1]


def _combine(partials):
    return pl.pallas_call(
        _add_body,
        grid=(10,),
        in_specs=[pl.BlockSpec((2, N_NODES // 10, D), lambda i: (0, i, 0))],
        out_specs=pl.BlockSpec((N_NODES // 10, D), lambda i: (i, 0)),
        out_shape=jax.ShapeDtypeStruct((N_NODES, D), jnp.float32),
    )(partials)


@functools.partial(
    pl.kernel,
    out_type=jax.ShapeDtypeStruct((NC, ACC_ROWS, D), jnp.float32),
    mesh=plsc.VectorSubcoreMesh(core_axis_name="c", subcore_axis_name="s"),
    scratch_types=[
        pltpu.VMEM_SHARED((ACC_ROWS, D), jnp.float32),
        pltpu.VMEM((CH, MICRO), jnp.int32),
        pltpu.VMEM((CH, MICRO), jnp.int32),
        pltpu.VMEM((NBUF, MICRO, D), jnp.float32),
        pltpu.SemaphoreType.DMA,
        pltpu.SemaphoreType.DMA,
    ],
)
def _sc_segment_sum(h_hbm, src_hbm, dst_hbm, out_hbm,
                    acc, srcv, dstv, bufs, s0, s1):
    sems = (s0, s1)
    c = lax.axis_index("c")
    s = lax.axis_index("s")
    wid = c * NS + s
    row0 = wid * M_PER_TILE

    zbuf = bufs.at[0]
    zero16 = jnp.zeros((16,), jnp.float32)

    def _zero_row(i, _):
        for cc in range(D // 16):
            zbuf[i, pl.ds(cc * 16, 16)] = zero16
        return 0

    lax.fori_loop(0, MICRO, _zero_row, 0)
    zbase = s * ZROWS
    for k in range(ZROWS // MICRO):
        pltpu.sync_copy(zbuf, acc.at[pl.ds(zbase + k * MICRO, MICRO)])

    plsc.subcore_barrier()

    def _chunk(k, _):
        pltpu.sync_copy(src_hbm.at[pl.ds(row0 + k * CH, CH)], srcv)
        pltpu.sync_copy(dst_hbm.at[pl.ds(row0 + k * CH, CH)], dstv)

        def _pair(j, _):
            descs = []
            for b in range(NBUF):
                m = j * NBUF + b
                descs.append(pltpu.async_copy(
                    h_hbm.at[srcv.at[m]], bufs.at[b], sems[b]))
            for b in range(NBUF):
                m = j * NBUF + b
                descs[b].wait()
                pltpu.sync_copy(bufs.at[b], acc.at[dstv.at[m]], add=True)
            return 0

        lax.fori_loop(0, CH // NBUF, _pair, 0)
        return 0

    lax.fori_loop(0, NCHUNK, _chunk, 0)

    plsc.subcore_barrier()

    obase = s * ZROWS
    pltpu.sync_copy(acc.at[pl.ds(obase, ZROWS)],
                    out_hbm.at[c, pl.ds(obase, ZROWS)])


def kernel(feat, edge_index, W, al, ar):
    del al, ar
    src = edge_index[0]
    dst = edge_index[1]
    pad = EP - N_EDGES
    iot = jax.lax.iota(jnp.int32, pad)
    pad_src = iot % N_NODES
    pad_dst = N_NODES + iot % (ACC_ROWS - N_NODES)
    src2d = jnp.concatenate([src, pad_src]).reshape(ROWS2D, MICRO)
    dst2d = jnp.concatenate([dst, pad_dst]).reshape(ROWS2D, MICRO)
    h = _project(feat, W)
    partials = _sc_segment_sum(h, src2d, dst2d)
    return _combine(partials)


# speedup vs baseline: 35.2505x; 1.1475x over previous
import functools

import jax
import jax.numpy as jnp
from jax import lax
from jax.experimental import pallas as pl
from jax.experimental.pallas import tpu as pltpu
from jax.experimental.pallas import tpu_sc as plsc

N_NODES = 10000
D = 128
N_EDGES = 320000

NC = 2
NS = 16
NW = NC * NS
MICRO = 128
NBUF = 2
M_PER_TILE = 80
CH = 8
NCHUNK = M_PER_TILE // CH
EP = NW * M_PER_TILE * MICRO
ROWS2D = EP // MICRO
ACC_ROWS = 10240
PAD_DST = ACC_ROWS - 1
ZROWS = ACC_ROWS // NS


def _mm_body(x_ref, w_ref, o_ref):
    o_ref[...] = jnp.dot(x_ref[...], w_ref[...],
                         preferred_element_type=jnp.float32)


def _project(feat, W):
    return pl.pallas_call(
        _mm_body,
        grid=(10,),
        in_specs=[
            pl.BlockSpec((N_NODES // 10, D), lambda i: (i, 0)),
            pl.BlockSpec((D, D), lambda i: (0, 0)),
        ],
        out_specs=pl.BlockSpec((N_NODES // 10, D), lambda i: (i, 0)),
        out_shape=jax.ShapeDtypeStruct((N_NODES, D), jnp.float32),
    )(feat, W)


def _add_body(p_ref, o_ref):
    o_ref[...] = p_ref[0] + p_ref[1]


def _combine(partials):
    return pl.pallas_call(
        _add_body,
        grid=(10,),
        in_specs=[pl.BlockSpec((2, N_NODES // 10, D), lambda i: (0, i, 0))],
        out_specs=pl.BlockSpec((N_NODES // 10, D), lambda i: (i, 0)),
        out_shape=jax.ShapeDtypeStruct((N_NODES, D), jnp.float32),
    )(partials)


@functools.partial(
    pl.kernel,
    out_type=jax.ShapeDtypeStruct((NC, ACC_ROWS, D), jnp.float32),
    mesh=plsc.VectorSubcoreMesh(core_axis_name="c", subcore_axis_name="s"),
    scratch_types=[
        pltpu.VMEM_SHARED((ACC_ROWS, D), jnp.float32),
        pltpu.VMEM((CH, MICRO), jnp.int32),
        pltpu.VMEM((CH, MICRO), jnp.int32),
        pltpu.VMEM((NBUF, MICRO, D), jnp.float32),
        pltpu.SemaphoreType.DMA,
        pltpu.SemaphoreType.DMA,
        pltpu.SemaphoreType.DMA,
        pltpu.SemaphoreType.DMA,
    ],
)
def _sc_segment_sum(h_hbm, src_hbm, dst_hbm, out_hbm,
                    acc, srcv, dstv, bufs, g0, g1, c0, c1):
    gsems = (g0, g1)
    ssems = (c0, c1)
    c = lax.axis_index("c")
    s = lax.axis_index("s")
    wid = c * NS + s
    row0 = wid * M_PER_TILE

    zbuf = bufs.at[0]
    zero16 = jnp.zeros((16,), jnp.float32)

    def _zero_row(i, _):
        for cc in range(D // 16):
            zbuf[i, pl.ds(cc * 16, 16)] = zero16
        return 0

    lax.fori_loop(0, MICRO, _zero_row, 0)
    zbase = s * ZROWS
    for k in range(ZROWS // MICRO):
        pltpu.sync_copy(zbuf, acc.at[pl.ds(zbase + k * MICRO, MICRO)])

    plsc.subcore_barrier()

    def _chunk(k, _):
        pltpu.sync_copy(src_hbm.at[pl.ds(row0 + k * CH, CH)], srcv)
        pltpu.sync_copy(dst_hbm.at[pl.ds(row0 + k * CH, CH)], dstv)

        gd = [
            pltpu.async_copy(h_hbm.at[srcv.at[b]], bufs.at[b], gsems[b])
            for b in range(NBUF)
        ]
        sd = [None] * NBUF
        for j in range(CH):
            b = j % NBUF
            gd[b].wait()
            sd[b] = pltpu.async_copy(
                bufs.at[b], acc.at[dstv.at[j]], ssems[b], add=True)
            if j + NBUF < CH:
                sd[b].wait()
                gd[b] = pltpu.async_copy(
                    h_hbm.at[srcv.at[j + NBUF]], bufs.at[b], gsems[b])
        for b in range(NBUF):
            sd[b].wait()
        return 0

    lax.fori_loop(0, NCHUNK, _chunk, 0)

    plsc.subcore_barrier()

    obase = s * ZROWS
    pltpu.sync_copy(acc.at[pl.ds(obase, ZROWS)],
                    out_hbm.at[c, pl.ds(obase, ZROWS)])


def kernel(feat, edge_index, W, al, ar):
    del al, ar
    src = edge_index[0]
    dst = edge_index[1]
    pad = EP - N_EDGES
    iot = jax.lax.iota(jnp.int32, pad)
    pad_src = iot % N_NODES
    pad_dst = N_NODES + iot % (ACC_ROWS - N_NODES)
    src2d = jnp.concatenate([src, pad_src]).reshape(ROWS2D, MICRO)
    dst2d = jnp.concatenate([dst, pad_dst]).reshape(ROWS2D, MICRO)
    h = _project(feat, W)
    partials = _sc_segment_sum(h, src2d, dst2d)
    return _combine(partials)


# speedup vs baseline: 36.7598x; 1.0428x over previous
import functools

import jax
import jax.numpy as jnp
from jax import lax
from jax.experimental import pallas as pl
from jax.experimental.pallas import tpu as pltpu
from jax.experimental.pallas import tpu_sc as plsc

N_NODES = 10000
D = 128
N_EDGES = 320000

NC = 2
NS = 16
NW = NC * NS
MICRO = 128
NBUF = 2
M_PER_TILE = 80
CH = 8
NCHUNK = M_PER_TILE // CH
EP = NW * M_PER_TILE * MICRO
ROWS2D = EP // MICRO
ACC_ROWS = 10240
PAD_DST = ACC_ROWS - 1
ZROWS = ACC_ROWS // NS


def _mm_body(x_ref, w_ref, o_ref):
    o_ref[...] = jnp.dot(x_ref[...], w_ref[...],
                         preferred_element_type=jnp.float32)


def _project(feat, W):
    return pl.pallas_call(
        _mm_body,
        grid=(10,),
        in_specs=[
            pl.BlockSpec((N_NODES // 10, D), lambda i: (i, 0)),
            pl.BlockSpec((D, D), lambda i: (0, 0)),
        ],
        out_specs=pl.BlockSpec((N_NODES // 10, D), lambda i: (i, 0)),
        out_shape=jax.ShapeDtypeStruct((N_NODES, D), jnp.float32),
    )(feat, W)


E_ROWS = N_EDGES // MICRO
E_ALIGNED = 2496


def _pad_body(e_ref, os_ref, od_ref):
    os_ref[0:E_ALIGNED] = e_ref[0, 0:E_ALIGNED]
    od_ref[0:E_ALIGNED] = e_ref[1, 0:E_ALIGNED]
    tail = ROWS2D - E_ALIGNED
    flat = (jax.lax.broadcasted_iota(jnp.int32, (tail - 4, MICRO), 0) * MICRO
            + jax.lax.broadcasted_iota(jnp.int32, (tail - 4, MICRO), 1))
    pad_src = flat % N_NODES
    pad_dst = N_NODES + jax.lax.rem(flat, ACC_ROWS - N_NODES)
    os_ref[E_ALIGNED:ROWS2D] = jnp.concatenate(
        [e_ref[0, E_ALIGNED:E_ROWS], pad_src], axis=0)
    od_ref[E_ALIGNED:ROWS2D] = jnp.concatenate(
        [e_ref[1, E_ALIGNED:E_ROWS], pad_dst], axis=0)


def _pad_edges(e3):
    return pl.pallas_call(
        _pad_body,
        in_specs=[pl.BlockSpec((2, E_ROWS, MICRO), lambda: (0, 0, 0))],
        out_specs=[
            pl.BlockSpec((ROWS2D, MICRO), lambda: (0, 0)),
            pl.BlockSpec((ROWS2D, MICRO), lambda: (0, 0)),
        ],
        out_shape=[
            jax.ShapeDtypeStruct((ROWS2D, MICRO), jnp.int32),
            jax.ShapeDtypeStruct((ROWS2D, MICRO), jnp.int32),
        ],
    )(e3)


def _add_body(p_ref, o_ref):
    o_ref[...] = p_ref[0] + p_ref[1]


def _combine(partials):
    return pl.pallas_call(
        _add_body,
        grid=(10,),
        in_specs=[pl.BlockSpec((2, N_NODES // 10, D), lambda i: (0, i, 0))],
        out_specs=pl.BlockSpec((N_NODES // 10, D), lambda i: (i, 0)),
        out_shape=jax.ShapeDtypeStruct((N_NODES, D), jnp.float32),
    )(partials)


@functools.partial(
    pl.kernel,
    out_type=jax.ShapeDtypeStruct((NC, ACC_ROWS, D), jnp.float32),
    mesh=plsc.VectorSubcoreMesh(core_axis_name="c", subcore_axis_name="s"),
    scratch_types=[
        pltpu.VMEM_SHARED((ACC_ROWS, D), jnp.float32),
        pltpu.VMEM((CH, MICRO), jnp.int32),
        pltpu.VMEM((CH, MICRO), jnp.int32),
        pltpu.VMEM((NBUF, MICRO, D), jnp.float32),
        pltpu.SemaphoreType.DMA,
        pltpu.SemaphoreType.DMA,
        pltpu.SemaphoreType.DMA,
        pltpu.SemaphoreType.DMA,
    ],
)
def _sc_segment_sum(h_hbm, src_hbm, dst_hbm, out_hbm,
                    acc, srcv, dstv, bufs, g0, g1, c0, c1):
    gsems = (g0, g1)
    ssems = (c0, c1)
    c = lax.axis_index("c")
    s = lax.axis_index("s")
    wid = c * NS + s
    row0 = wid * M_PER_TILE

    zbuf = bufs.at[0]
    zero16 = jnp.zeros((16,), jnp.float32)

    def _zero_row(i, _):
        for cc in range(D // 16):
            zbuf[i, pl.ds(cc * 16, 16)] = zero16
        return 0

    lax.fori_loop(0, MICRO, _zero_row, 0)
    zbase = s * ZROWS
    for k in range(ZROWS // MICRO):
        pltpu.sync_copy(zbuf, acc.at[pl.ds(zbase + k * MICRO, MICRO)])

    plsc.subcore_barrier()

    def _chunk(k, _):
        pltpu.sync_copy(src_hbm.at[pl.ds(row0 + k * CH, CH)], srcv)
        pltpu.sync_copy(dst_hbm.at[pl.ds(row0 + k * CH, CH)], dstv)

        gd = [
            pltpu.async_copy(h_hbm.at[srcv.at[b]], bufs.at[b], gsems[b])
            for b in range(NBUF)
        ]
        sd = [None] * NBUF
        for j in range(CH):
            b = j % NBUF
            gd[b].wait()
            sd[b] = pltpu.async_copy(
                bufs.at[b], acc.at[dstv.at[j]], ssems[b], add=True)
            if j + NBUF < CH:
                sd[b].wait()
                gd[b] = pltpu.async_copy(
                    h_hbm.at[srcv.at[j + NBUF]], bufs.at[b], gsems[b])
        for b in range(NBUF):
            sd[b].wait()
        return 0

    lax.fori_loop(0, NCHUNK, _chunk, 0)

    plsc.subcore_barrier()

    obase = s * ZROWS
    pltpu.sync_copy(acc.at[pl.ds(obase, ZROWS)],
                    out_hbm.at[c, pl.ds(obase, ZROWS)])


def kernel(feat, edge_index, W, al, ar):
    del al, ar
    src2d, dst2d = _pad_edges(edge_index.reshape(2, E_ROWS, MICRO))
    h = _project(feat, W)
    partials = _sc_segment_sum(h, src2d, dst2d)
    return _combine(partials)


# speedup vs baseline: 40.4109x; 1.0993x over previous
import functools

import jax
import jax.numpy as jnp
from jax import lax
from jax.experimental import pallas as pl
from jax.experimental.pallas import tpu as pltpu
from jax.experimental.pallas import tpu_sc as plsc

N_NODES = 10000
D = 128
N_EDGES = 320000

NC = 2
NS = 16
NW = NC * NS
MICRO = 128
NBUF = 2
M_PER_TILE = 80
CH = 8
SUPER = 16
NSUPER = M_PER_TILE // SUPER
EP = NW * M_PER_TILE * MICRO
ROWS2D = EP // MICRO
OCTETS = ROWS2D // CH
OCT_PER_TILE = M_PER_TILE // CH
ACC_ROWS = 10240
ZROWS = ACC_ROWS // NS
E_ROWS = N_EDGES // MICRO
E_ALIGNED = 2496


def _pad_body(e_ref, o_ref):
    tail = ROWS2D - E_ALIGNED
    flat = (jax.lax.broadcasted_iota(jnp.int32, (tail - 4, MICRO), 0) * MICRO
            + jax.lax.broadcasted_iota(jnp.int32, (tail - 4, MICRO), 1))
    pad_src = flat % N_NODES
    pad_dst = N_NODES + jax.lax.rem(flat, ACC_ROWS - N_NODES)
    src = jnp.concatenate(
        [e_ref[0, 0:E_ALIGNED], e_ref[0, E_ALIGNED:E_ROWS], pad_src], axis=0)
    dst = jnp.concatenate(
        [e_ref[1, 0:E_ALIGNED], e_ref[1, E_ALIGNED:E_ROWS], pad_dst], axis=0)
    o_ref[:, 0] = src.reshape(OCTETS, CH, MICRO)
    o_ref[:, 1] = dst.reshape(OCTETS, CH, MICRO)


def _pad_edges(e3):
    return pl.pallas_call(
        _pad_body,
        in_specs=[pl.BlockSpec((2, E_ROWS, MICRO), lambda: (0, 0, 0))],
        out_specs=pl.BlockSpec((OCTETS, 2, CH, MICRO), lambda: (0, 0, 0, 0)),
        out_shape=jax.ShapeDtypeStruct((OCTETS, 2, CH, MICRO), jnp.int32),
    )(e3)


def _mm_body(x_ref, w_ref, o_ref):
    o_ref[...] = jnp.dot(x_ref[...], w_ref[...],
                         preferred_element_type=jnp.float32)


def _project(feat, W):
    return pl.pallas_call(
        _mm_body,
        grid=(10,),
        in_specs=[
            pl.BlockSpec((N_NODES // 10, D), lambda i: (i, 0)),
            pl.BlockSpec((D, D), lambda i: (0, 0)),
        ],
        out_specs=pl.BlockSpec((N_NODES // 10, D), lambda i: (i, 0)),
        out_shape=jax.ShapeDtypeStruct((N_NODES, D), jnp.float32),
    )(feat, W)


def _add_body(p_ref, o_ref):
    o_ref[...] = p_ref[0] + p_ref[1]


def _combine(partials):
    return pl.pallas_call(
        _add_body,
        grid=(10,),
        in_specs=[pl.BlockSpec((2, N_NODES // 10, D), lambda i: (0, i, 0))],
        out_specs=pl.BlockSpec((N_NODES // 10, D), lambda i: (i, 0)),
        out_shape=jax.ShapeDtypeStruct((N_NODES, D), jnp.float32),
    )(partials)


@functools.partial(
    pl.kernel,
    out_type=jax.ShapeDtypeStruct((NC, ACC_ROWS, D), jnp.float32),
    mesh=plsc.VectorSubcoreMesh(core_axis_name="c", subcore_axis_name="s"),
    scratch_types=[
        pltpu.VMEM_SHARED((ACC_ROWS, D), jnp.float32),
        pltpu.VMEM((2, 2, CH, MICRO), jnp.int32),
        pltpu.VMEM((NBUF, MICRO, D), jnp.float32),
        pltpu.SemaphoreType.DMA,
        pltpu.SemaphoreType.DMA,
        pltpu.SemaphoreType.DMA,
        pltpu.SemaphoreType.DMA,
        pltpu.SemaphoreType.DMA,
        pltpu.SemaphoreType.DMA,
    ],
)
def _sc_segment_sum(h_hbm, idx_hbm, out_hbm,
                    acc, idxv, bufs, g0, g1, c0, c1, i0, i1):
    gsems = (g0, g1)
    ssems = (c0, c1)
    isems = (i0, i1)
    c = lax.axis_index("c")
    s = lax.axis_index("s")
    wid = c * NS + s
    oct0 = wid * OCT_PER_TILE

    def _stage(k, p):
        pltpu.async_copy(idx_hbm.at[oct0 + k], idxv.at[p], isems[p])

    def _stage_wait(p):
        pltpu.make_async_copy(idx_hbm.at[0], idxv.at[p], isems[p]).wait()

    _stage(0, 0)
    _stage(1, 1)

    zbuf = bufs.at[0]
    zero16 = jnp.zeros((16,), jnp.float32)

    def _zero_row(i, _):
        for cc in range(D // 16):
            zbuf[i, pl.ds(cc * 16, 16)] = zero16
        return 0

    lax.fori_loop(0, MICRO, _zero_row, 0)
    zbase = s * ZROWS
    for k in range(ZROWS // MICRO):
        pltpu.sync_copy(zbuf, acc.at[pl.ds(zbase + k * MICRO, MICRO)])

    plsc.subcore_barrier()

    def _super(K, _):
        _stage_wait(0)
        gd = [
            pltpu.async_copy(
                h_hbm.at[idxv.at[b // CH, 0, b % CH]], bufs.at[b], gsems[b])
            for b in range(NBUF)
        ]
        sd = [None] * NBUF
        for j in range(SUPER):
            b = j % NBUF
            gd[b].wait()
            sd[b] = pltpu.async_copy(
                bufs.at[b], acc.at[idxv.at[j // CH, 1, j % CH]], ssems[b],
                add=True)
            if j == CH - NBUF - 1:
                _stage_wait(1)
            if j + NBUF < SUPER:
                sd[b].wait()
                jn = j + NBUF
                gd[b] = pltpu.async_copy(
                    h_hbm.at[idxv.at[jn // CH, 0, jn % CH]], bufs.at[b],
                    gsems[b])
        for b in range(NBUF):
            sd[b].wait()

        @pl.when(K < NSUPER - 1)
        def _prefetch():
            _stage(2 * K + 2, 0)
            _stage(2 * K + 3, 1)

        return 0

    lax.fori_loop(0, NSUPER, _super, 0)

    plsc.subcore_barrier()

    obase = s * ZROWS
    pltpu.sync_copy(acc.at[pl.ds(obase, ZROWS)],
                    out_hbm.at[c, pl.ds(obase, ZROWS)])


def kernel(feat, edge_index, W, al, ar):
    del al, ar
    idx = _pad_edges(edge_index.reshape(2, E_ROWS, MICRO))
    h = _project(feat, W)
    partials = _sc_segment_sum(h, idx)
    return _combine(partials)


# speedup vs baseline: 41.7913x; 1.0342x over previous
import functools

import jax
import jax.numpy as jnp
from jax import lax
from jax.experimental import pallas as pl
from jax.experimental.pallas import tpu as pltpu
from jax.experimental.pallas import tpu_sc as plsc

N_NODES = 10000
D = 128
N_EDGES = 320000

NC = 2
NS = 16
NW = NC * NS
MICRO = 128
NBUF = 2
M_PER_TILE = 80
CH = 8
SUPER = 16
NSUPER = M_PER_TILE // SUPER
EP = NW * M_PER_TILE * MICRO
ROWS2D = EP // MICRO
OCTETS = ROWS2D // CH
OCT_PER_TILE = M_PER_TILE // CH
ACC_ROWS = 10240
ZROWS = ACC_ROWS // NS
E_ROWS = N_EDGES // MICRO
E_ALIGNED = 2496


def _pad_body(e_ref, o_ref):
    tail = ROWS2D - E_ROWS
    flat = (jax.lax.broadcasted_iota(jnp.int32, (tail, MICRO), 0) * MICRO
            + jax.lax.broadcasted_iota(jnp.int32, (tail, MICRO), 1))
    pad_src = flat % N_NODES
    pad_dst = N_NODES + jax.lax.rem(flat, ACC_ROWS - N_NODES)
    src = jnp.concatenate(
        [e_ref[0].reshape(E_ROWS, MICRO), pad_src], axis=0)
    dst = jnp.concatenate(
        [e_ref[1].reshape(E_ROWS, MICRO), pad_dst], axis=0)
    o_ref[:, 0] = src.reshape(OCTETS, CH, MICRO)
    o_ref[:, 1] = dst.reshape(OCTETS, CH, MICRO)


def _pad_edges(edge_index):
    return pl.pallas_call(
        _pad_body,
        in_specs=[pl.BlockSpec((2, N_EDGES), lambda: (0, 0))],
        out_specs=pl.BlockSpec((OCTETS, 2, CH, MICRO), lambda: (0, 0, 0, 0)),
        out_shape=jax.ShapeDtypeStruct((OCTETS, 2, CH, MICRO), jnp.int32),
    )(edge_index)


def _mm_body(x_ref, w_ref, o_ref):
    o_ref[...] = jnp.dot(x_ref[...].astype(jnp.bfloat16),
                         w_ref[...].astype(jnp.bfloat16),
                         preferred_element_type=jnp.float32)


def _project(feat, W):
    return pl.pallas_call(
        _mm_body,
        grid=(10,),
        in_specs=[
            pl.BlockSpec((N_NODES // 10, D), lambda i: (i, 0)),
            pl.BlockSpec((D, D), lambda i: (0, 0)),
        ],
        out_specs=pl.BlockSpec((N_NODES // 10, D), lambda i: (i, 0)),
        out_shape=jax.ShapeDtypeStruct((N_NODES, D), jnp.float32),
    )(feat, W)


def _add_body(p_ref, o_ref):
    o_ref[...] = p_ref[0] + p_ref[1]


def _combine(partials):
    return pl.pallas_call(
        _add_body,
        grid=(10,),
        in_specs=[pl.BlockSpec((2, N_NODES // 10, D), lambda i: (0, i, 0))],
        out_specs=pl.BlockSpec((N_NODES // 10, D), lambda i: (i, 0)),
        out_shape=jax.ShapeDtypeStruct((N_NODES, D), jnp.float32),
    )(partials)


@functools.partial(
    pl.kernel,
    out_type=jax.ShapeDtypeStruct((NC, ACC_ROWS, D), jnp.float32),
    mesh=plsc.VectorSubcoreMesh(core_axis_name="c", subcore_axis_name="s"),
    scratch_types=[
        pltpu.VMEM_SHARED((ACC_ROWS, D), jnp.float32),
        pltpu.VMEM((2, 2, CH, MICRO), jnp.int32),
        pltpu.VMEM((NBUF, MICRO, D), jnp.float32),
        pltpu.SemaphoreType.DMA,
        pltpu.SemaphoreType.DMA,
        pltpu.SemaphoreType.DMA,
        pltpu.SemaphoreType.DMA,
        pltpu.SemaphoreType.DMA,
        pltpu.SemaphoreType.DMA,
    ],
)
def _sc_segment_sum(h_hbm, idx_hbm, out_hbm,
                    acc, idxv, bufs, g0, g1, c0, c1, i0, i1):
    gsems = (g0, g1)
    ssems = (c0, c1)
    isems = (i0, i1)
    c = lax.axis_index("c")
    s = lax.axis_index("s")
    wid = c * NS + s
    oct0 = wid * OCT_PER_TILE

    def _stage(k, p):
        pltpu.async_copy(idx_hbm.at[oct0 + k], idxv.at[p], isems[p])

    def _stage_wait(p):
        pltpu.make_async_copy(idx_hbm.at[0], idxv.at[p], isems[p]).wait()

    _stage(0, 0)
    _stage(1, 1)

    zbuf = bufs.at[0]
    zero16 = jnp.zeros((16,), jnp.float32)

    def _zero_row(i, _):
        for cc in range(D // 16):
            zbuf[i, pl.ds(cc * 16, 16)] = zero16
        return 0

    lax.fori_loop(0, MICRO, _zero_row, 0)
    zbase = s * ZROWS
    for k in range(ZROWS // MICRO):
        pltpu.sync_copy(zbuf, acc.at[pl.ds(zbase + k * MICRO, MICRO)])

    plsc.subcore_barrier()

    def _super(K, _):
        _stage_wait(0)
        gd = [
            pltpu.async_copy(
                h_hbm.at[idxv.at[b // CH, 0, b % CH]], bufs.at[b], gsems[b])
            for b in range(NBUF)
        ]
        sd = [None] * NBUF
        for j in range(SUPER):
            b = j % NBUF
            gd[b].wait()
            sd[b] = pltpu.async_copy(
                bufs.at[b], acc.at[idxv.at[j // CH, 1, j % CH]], ssems[b],
                add=True)
            if j == CH - NBUF - 1:
                _stage_wait(1)
            if j + NBUF < SUPER:
                sd[b].wait()
                jn = j + NBUF
                gd[b] = pltpu.async_copy(
                    h_hbm.at[idxv.at[jn // CH, 0, jn % CH]], bufs.at[b],
                    gsems[b])
        for b in range(NBUF):
            sd[b].wait()

        @pl.when(K < NSUPER - 1)
        def _prefetch():
            _stage(2 * K + 2, 0)
            _stage(2 * K + 3, 1)

        return 0

    lax.fori_loop(0, NSUPER, _super, 0)

    plsc.subcore_barrier()

    obase = s * ZROWS
    pltpu.sync_copy(acc.at[pl.ds(obase, ZROWS)],
                    out_hbm.at[c, pl.ds(obase, ZROWS)])


def kernel(feat, edge_index, W, al, ar):
    del al, ar
    idx = _pad_edges(edge_index)
    h = _project(feat, W)
    partials = _sc_segment_sum(h, idx)
    return _combine(partials)


# speedup vs baseline: 43.1151x; 1.0317x over previous
import functools

import jax
import jax.numpy as jnp
from jax import lax
from jax.experimental import pallas as pl
from jax.experimental.pallas import tpu as pltpu
from jax.experimental.pallas import tpu_sc as plsc

N_NODES = 10000
D = 128
N_EDGES = 320000

NC = 2
NS = 16
NW = NC * NS
MICRO = 128
NBUF = 2
M_PER_TILE = 80
CH = 8
SUPER = 16
NSUPER = M_PER_TILE // SUPER
EP = NW * M_PER_TILE * MICRO
ROWS2D = EP // MICRO
OCTETS = ROWS2D // CH
OCT_PER_TILE = M_PER_TILE // CH
ACC_ROWS = 10240
ZROWS = ACC_ROWS // NS
E_ROWS = N_EDGES // MICRO
E_ALIGNED = 2496


def _pad_body(e_ref, o_ref):
    tail = ROWS2D - E_ROWS
    flat = (jax.lax.broadcasted_iota(jnp.int32, (tail, MICRO), 0) * MICRO
            + jax.lax.broadcasted_iota(jnp.int32, (tail, MICRO), 1))
    pad_src = flat % N_NODES
    pad_dst = N_NODES + jax.lax.rem(flat, ACC_ROWS - N_NODES)
    src = jnp.concatenate(
        [e_ref[0].reshape(E_ROWS, MICRO), pad_src], axis=0)
    dst = jnp.concatenate(
        [e_ref[1].reshape(E_ROWS, MICRO), pad_dst], axis=0)
    o_ref[:, 0] = src.reshape(OCTETS, CH, MICRO)
    o_ref[:, 1] = dst.reshape(OCTETS, CH, MICRO)


def _pad_edges(edge_index):
    return pl.pallas_call(
        _pad_body,
        in_specs=[pl.BlockSpec((2, N_EDGES), lambda: (0, 0))],
        out_specs=pl.BlockSpec((OCTETS, 2, CH, MICRO), lambda: (0, 0, 0, 0)),
        out_shape=jax.ShapeDtypeStruct((OCTETS, 2, CH, MICRO), jnp.int32),
    )(edge_index)


def _mm_body(x_ref, w_ref, o_ref):
    o_ref[...] = jnp.dot(x_ref[...], w_ref[...],
                         preferred_element_type=jnp.float32)


def _project(feat, W):
    return pl.pallas_call(
        _mm_body,
        grid=(5,),
        in_specs=[
            pl.BlockSpec((N_NODES // 5, D), lambda i: (i, 0)),
            pl.BlockSpec((D, D), lambda i: (0, 0)),
        ],
        out_specs=pl.BlockSpec((N_NODES // 5, D), lambda i: (i, 0)),
        out_shape=jax.ShapeDtypeStruct((N_NODES, D), jnp.float32),
    )(feat, W)


def _add_body(p_ref, o_ref):
    o_ref[...] = p_ref[0] + p_ref[1]


def _combine(partials):
    return pl.pallas_call(
        _add_body,
        grid=(10,),
        in_specs=[pl.BlockSpec((2, N_NODES // 10, D), lambda i: (0, i, 0))],
        out_specs=pl.BlockSpec((N_NODES // 10, D), lambda i: (i, 0)),
        out_shape=jax.ShapeDtypeStruct((N_NODES, D), jnp.float32),
    )(partials)


@functools.partial(
    pl.kernel,
    out_type=jax.ShapeDtypeStruct((NC, ACC_ROWS, D), jnp.float32),
    mesh=plsc.VectorSubcoreMesh(core_axis_name="c", subcore_axis_name="s"),
    scratch_types=[
        pltpu.VMEM_SHARED((ACC_ROWS, D), jnp.float32),
        pltpu.VMEM((2, 2, CH, MICRO), jnp.int32),
        pltpu.VMEM((NBUF, MICRO, D), jnp.float32),
        pltpu.SemaphoreType.DMA,
        pltpu.SemaphoreType.DMA,
        pltpu.SemaphoreType.DMA,
        pltpu.SemaphoreType.DMA,
        pltpu.SemaphoreType.DMA,
        pltpu.SemaphoreType.DMA,
    ],
)
def _sc_segment_sum(h_hbm, idx_hbm, out_hbm,
                    acc, idxv, bufs, g0, g1, c0, c1, i0, i1):
    gsems = (g0, g1)
    ssems = (c0, c1)
    isems = (i0, i1)
    c = lax.axis_index("c")
    s = lax.axis_index("s")
    wid = c * NS + s
    oct0 = wid * OCT_PER_TILE

    def _stage(k, p):
        pltpu.async_copy(idx_hbm.at[oct0 + k], idxv.at[p], isems[p])

    def _stage_wait(p):
        pltpu.make_async_copy(idx_hbm.at[0], idxv.at[p], isems[p]).wait()

    _stage(0, 0)
    _stage(1, 1)

    zbuf = bufs.at[0]
    zero16 = jnp.zeros((16,), jnp.float32)

    def _zero_row(i, _):
        for cc in range(D // 16):
            zbuf[i, pl.ds(cc * 16, 16)] = zero16
        return 0

    lax.fori_loop(0, MICRO, _zero_row, 0)
    zbase = s * ZROWS
    for k in range(ZROWS // MICRO):
        pltpu.sync_copy(zbuf, acc.at[pl.ds(zbase + k * MICRO, MICRO)])

    plsc.subcore_barrier()

    def _super(K, _):
        _stage_wait(0)
        gd = [
            pltpu.async_copy(
                h_hbm.at[idxv.at[b // CH, 0, b % CH]], bufs.at[b], gsems[b])
            for b in range(NBUF)
        ]
        sd = [None] * NBUF
        for j in range(SUPER):
            b = j % NBUF
            gd[b].wait()
            sd[b] = pltpu.async_copy(
                bufs.at[b], acc.at[idxv.at[j // CH, 1, j % CH]], ssems[b],
                add=True)
            if j == CH - NBUF - 1:
                _stage_wait(1)
            if j == CH:
                @pl.when(K < NSUPER - 1)
                def _pf0():
                    _stage(2 * K + 2, 0)
            if j + NBUF < SUPER:
                sd[b].wait()
                jn = j + NBUF
                gd[b] = pltpu.async_copy(
                    h_hbm.at[idxv.at[jn // CH, 0, jn % CH]], bufs.at[b],
                    gsems[b])
        for b in range(NBUF):
            sd[b].wait()

        @pl.when(K < NSUPER - 1)
        def _prefetch():
            _stage(2 * K + 3, 1)

        return 0

    lax.fori_loop(0, NSUPER, _super, 0)

    plsc.subcore_barrier()

    obase = s * ZROWS
    pltpu.sync_copy(acc.at[pl.ds(obase, ZROWS)],
                    out_hbm.at[c, pl.ds(obase, ZROWS)])


def kernel(feat, edge_index, W, al, ar):
    del al, ar
    idx = _pad_edges(edge_index)
    h = _project(feat, W)
    partials = _sc_segment_sum(h, idx)
    return _combine(partials)


# speedup vs baseline: 43.2831x; 1.0039x over previous
import functools

import jax
import jax.numpy as jnp
from jax import lax
from jax.experimental import pallas as pl
from jax.experimental.pallas import tpu as pltpu
from jax.experimental.pallas import tpu_sc as plsc

N_NODES = 10000
D = 128
N_EDGES = 320000

NC = 2
NS = 16
NW = NC * NS
MICRO = 128
NBUF = 2
M_PER_TILE = 80
CH = 8
SUPER = 16
NSUPER = M_PER_TILE // SUPER
EP = NW * M_PER_TILE * MICRO
ROWS2D = EP // MICRO
OCTETS = ROWS2D // CH
OCT_PER_TILE = M_PER_TILE // CH
ACC_ROWS = 10240
ZROWS = ACC_ROWS // NS
E_ROWS = N_EDGES // MICRO
E_ALIGNED = 2496


def _pad_body(e_ref, o_ref):
    tail = ROWS2D - E_ROWS
    flat = (jax.lax.broadcasted_iota(jnp.int32, (tail, MICRO), 0) * MICRO
            + jax.lax.broadcasted_iota(jnp.int32, (tail, MICRO), 1))
    pad_src = flat % N_NODES
    pad_dst = N_NODES + jax.lax.rem(flat, ACC_ROWS - N_NODES)
    src = jnp.concatenate(
        [e_ref[0].reshape(E_ROWS, MICRO), pad_src], axis=0)
    dst = jnp.concatenate(
        [e_ref[1].reshape(E_ROWS, MICRO), pad_dst], axis=0)
    o_ref[:, 0] = src.reshape(OCTETS, CH, MICRO)
    o_ref[:, 1] = dst.reshape(OCTETS, CH, MICRO)


def _pad_edges(edge_index):
    return pl.pallas_call(
        _pad_body,
        in_specs=[pl.BlockSpec((2, N_EDGES), lambda: (0, 0))],
        out_specs=pl.BlockSpec((OCTETS, 2, CH, MICRO), lambda: (0, 0, 0, 0)),
        out_shape=jax.ShapeDtypeStruct((OCTETS, 2, CH, MICRO), jnp.int32),
    )(edge_index)


def _mm_body(x_ref, w_ref, o_ref):
    o_ref[...] = jnp.dot(x_ref[...], w_ref[...],
                         preferred_element_type=jnp.float32)


def _project(feat, W):
    return pl.pallas_call(
        _mm_body,
        grid=(5,),
        in_specs=[
            pl.BlockSpec((N_NODES // 5, D), lambda i: (i, 0)),
            pl.BlockSpec((D, D), lambda i: (0, 0)),
        ],
        out_specs=pl.BlockSpec((N_NODES // 5, D), lambda i: (i, 0)),
        out_shape=jax.ShapeDtypeStruct((N_NODES, D), jnp.float32),
    )(feat, W)


def _add_body(p_ref, o_ref):
    o_ref[...] = p_ref[0] + p_ref[1]


def _combine(partials):
    return pl.pallas_call(
        _add_body,
        grid=(10,),
        in_specs=[pl.BlockSpec((2, N_NODES // 10, D), lambda i: (0, i, 0))],
        out_specs=pl.BlockSpec((N_NODES // 10, D), lambda i: (i, 0)),
        out_shape=jax.ShapeDtypeStruct((N_NODES, D), jnp.float32),
    )(partials)


@functools.partial(
    pl.kernel,
    out_type=jax.ShapeDtypeStruct((NC, ACC_ROWS, D), jnp.float32),
    mesh=plsc.VectorSubcoreMesh(core_axis_name="c", subcore_axis_name="s"),
    scratch_types=[
        pltpu.VMEM_SHARED((ACC_ROWS, D), jnp.float32),
        pltpu.VMEM((2, 2, CH, MICRO), jnp.int32),
        pltpu.VMEM((NBUF, MICRO, D), jnp.float32),
        pltpu.SemaphoreType.DMA,
        pltpu.SemaphoreType.DMA,
        pltpu.SemaphoreType.DMA,
        pltpu.SemaphoreType.DMA,
        pltpu.SemaphoreType.DMA,
        pltpu.SemaphoreType.DMA,
    ],
)
def _sc_segment_sum(h_hbm, idx_hbm, out_hbm,
                    acc, idxv, bufs, g0, g1, c0, c1, i0, i1):
    gsems = (g0, g1)
    ssems = (c0, c1)
    isems = (i0, i1)
    c = lax.axis_index("c")
    s = lax.axis_index("s")
    wid = c * NS + s
    oct0 = wid * OCT_PER_TILE

    def _stage(k, p):
        pltpu.async_copy(idx_hbm.at[oct0 + k], idxv.at[p], isems[p])

    def _stage_wait(p):
        pltpu.make_async_copy(idx_hbm.at[0], idxv.at[p], isems[p]).wait()

    _stage(0, 0)
    _stage(1, 1)

    zbuf = bufs.at[0]
    zero16 = jnp.zeros((16,), jnp.float32)

    def _zero_row(i, _):
        for cc in range(D // 16):
            zbuf[i, pl.ds(cc * 16, 16)] = zero16
        return 0

    lax.fori_loop(0, MICRO, _zero_row, 0)
    zbase = s * ZROWS
    zd = [
        pltpu.async_copy(zbuf, acc.at[pl.ds(zbase + k * MICRO, MICRO)],
                         gsems[k % NBUF])
        for k in range(ZROWS // MICRO)
    ]
    for d in zd:
        d.wait()

    plsc.subcore_barrier()

    def _super(K, _):
        _stage_wait(0)
        gd = [
            pltpu.async_copy(
                h_hbm.at[idxv.at[b // CH, 0, b % CH]], bufs.at[b], gsems[b])
            for b in range(NBUF)
        ]
        sd = [None] * NBUF
        for j in range(SUPER):
            b = j % NBUF
            gd[b].wait()
            sd[b] = pltpu.async_copy(
                bufs.at[b], acc.at[idxv.at[j // CH, 1, j % CH]], ssems[b],
                add=True)
            if j == CH - NBUF - 1:
                _stage_wait(1)
            if j == CH:
                @pl.when(K < NSUPER - 1)
                def _pf0():
                    _stage(2 * K + 2, 0)
            if j + NBUF < SUPER:
                sd[b].wait()
                jn = j + NBUF
                gd[b] = pltpu.async_copy(
                    h_hbm.at[idxv.at[jn // CH, 0, jn % CH]], bufs.at[b],
                    gsems[b])
        for b in range(NBUF):
            sd[b].wait()

        @pl.when(K < NSUPER - 1)
        def _prefetch():
            _stage(2 * K + 3, 1)

        return 0

    lax.fori_loop(0, NSUPER, _super, 0)

    plsc.subcore_barrier()

    obase = s * ZROWS
    pltpu.sync_copy(acc.at[pl.ds(obase, ZROWS)],
                    out_hbm.at[c, pl.ds(obase, ZROWS)])


def kernel(feat, edge_index, W, al, ar):
    del al, ar
    idx = _pad_edges(edge_index)
    h = _project(feat, W)
    partials = _sc_segment_sum(h, idx)
    return _combine(partials)


# speedup vs baseline: 44.8110x; 1.0353x over previous
import functools

import jax
import jax.numpy as jnp
from jax import lax
from jax.experimental import pallas as pl
from jax.experimental.pallas import tpu as pltpu
from jax.experimental.pallas import tpu_sc as plsc

N_NODES = 10000
D = 128
N_EDGES = 320000

NC = 2
NS = 16
NW = NC * NS
MICRO = 128
NBUF = 2
M_PER_TILE = 80
CH = 8
SUPER = 16
NSUPER = M_PER_TILE // SUPER
EP = NW * M_PER_TILE * MICRO
ROWS2D = EP // MICRO
OCTETS = ROWS2D // CH
OCT_PER_TILE = M_PER_TILE // CH
ACC_ROWS = 10240
ZROWS = ACC_ROWS // NS
E_ROWS = N_EDGES // MICRO
E_ALIGNED = 2496


def _pad_body(e_ref, o_ref):
    tail = ROWS2D - E_ROWS
    flat = (jax.lax.broadcasted_iota(jnp.int32, (tail, MICRO), 0) * MICRO
            + jax.lax.broadcasted_iota(jnp.int32, (tail, MICRO), 1))
    pad_src = flat % N_NODES
    pad_dst = N_NODES + jax.lax.rem(flat, ACC_ROWS - N_NODES)
    src = jnp.concatenate(
        [e_ref[0].reshape(E_ROWS, MICRO), pad_src], axis=0)
    dst = jnp.concatenate(
        [e_ref[1].reshape(E_ROWS, MICRO), pad_dst], axis=0)
    o_ref[:, 0] = src.reshape(OCTETS, CH, MICRO)
    o_ref[:, 1] = dst.reshape(OCTETS, CH, MICRO)


def _pad_edges(edge_index):
    return pl.pallas_call(
        _pad_body,
        in_specs=[pl.BlockSpec((2, N_EDGES), lambda: (0, 0))],
        out_specs=pl.BlockSpec((OCTETS, 2, CH, MICRO), lambda: (0, 0, 0, 0)),
        out_shape=jax.ShapeDtypeStruct((OCTETS, 2, CH, MICRO), jnp.int32),
    )(edge_index)


def _mm_body(x_ref, w_ref, o_ref):
    o_ref[...] = jnp.dot(x_ref[...], w_ref[...],
                         preferred_element_type=jnp.float32)


def _project(feat, W):
    return pl.pallas_call(
        _mm_body,
        grid=(5,),
        in_specs=[
            pl.BlockSpec((N_NODES // 5, D), lambda i: (i, 0)),
            pl.BlockSpec((D, D), lambda i: (0, 0)),
        ],
        out_specs=pl.BlockSpec((N_NODES // 5, D), lambda i: (i, 0)),
        out_shape=jax.ShapeDtypeStruct((N_NODES, D), jnp.float32),
    )(feat, W)


def _add_body(p_ref, o_ref):
    o_ref[...] = p_ref[0] + p_ref[1]


def _combine(partials):
    return pl.pallas_call(
        _add_body,
        grid=(10,),
        in_specs=[pl.BlockSpec((2, N_NODES // 10, D), lambda i: (0, i, 0))],
        out_specs=pl.BlockSpec((N_NODES // 10, D), lambda i: (i, 0)),
        out_shape=jax.ShapeDtypeStruct((N_NODES, D), jnp.float32),
    )(partials)


@functools.partial(
    pl.kernel,
    out_type=jax.ShapeDtypeStruct((NC, ACC_ROWS, D), jnp.float32),
    mesh=plsc.VectorSubcoreMesh(core_axis_name="c", subcore_axis_name="s"),
    scratch_types=[
        pltpu.VMEM_SHARED((ACC_ROWS, D), jnp.float32),
        pltpu.VMEM((2, 2, CH, MICRO), jnp.int32),
        pltpu.VMEM((NBUF, MICRO, D), jnp.float32),
        pltpu.SemaphoreType.DMA,
        pltpu.SemaphoreType.DMA,
        pltpu.SemaphoreType.DMA,
        pltpu.SemaphoreType.DMA,
        pltpu.SemaphoreType.DMA,
        pltpu.SemaphoreType.DMA,
    ],
)
def _sc_segment_sum(h_hbm, idx_hbm, out_hbm,
                    acc, idxv, bufs, g0, g1, c0, c1, i0, i1):
    gsems = (g0, g1)
    ssems = (c0, c1)
    isems = (i0, i1)
    c = lax.axis_index("c")
    s = lax.axis_index("s")
    wid = c * NS + s
    oct0 = wid * OCT_PER_TILE

    def _stage(k, p):
        pltpu.async_copy(idx_hbm.at[oct0 + k], idxv.at[p], isems[p])

    def _stage_wait(p):
        pltpu.make_async_copy(idx_hbm.at[0], idxv.at[p], isems[p]).wait()

    def _fire_gather(j, b):
        m = j % SUPER
        pltpu.async_copy(
            h_hbm.at[idxv.at[m // CH, 0, m % CH]], bufs.at[b], gsems[b])

    def _gather_wait(b):
        pltpu.make_async_copy(
            h_hbm.at[pl.ds(0, MICRO)], bufs.at[b], gsems[b]).wait()

    def _scatter_wait(b):
        pltpu.make_async_copy(
            bufs.at[b], acc.at[pl.ds(0, MICRO)], ssems[b]).wait()

    _stage(0, 0)
    _stage(1, 1)

    zbuf = bufs.at[0]
    zero16 = jnp.zeros((16,), jnp.float32)

    def _zero_row(i, _):
        for cc in range(D // 16):
            zbuf[i, pl.ds(cc * 16, 16)] = zero16
        return 0

    lax.fori_loop(0, MICRO, _zero_row, 0)
    zbase = s * ZROWS
    zd = [
        pltpu.async_copy(zbuf, acc.at[pl.ds(zbase + k * MICRO, MICRO)],
                         gsems[k % NBUF])
        for k in range(ZROWS // MICRO)
    ]
    for d in zd:
        d.wait()

    plsc.subcore_barrier()

    _stage_wait(0)
    _fire_gather(0, 0)
    _fire_gather(1, 1)

    def _super(K, _):
        not_last = K < NSUPER - 1
        for j in range(SUPER):
            b = j % NBUF
            _gather_wait(b)
            pltpu.async_copy(
                bufs.at[b], acc.at[idxv.at[j // CH, 1, j % CH]], ssems[b],
                add=True)
            if j == CH - NBUF:
                _stage_wait(1)
            if j == CH:
                @pl.when(not_last)
                def _pf0():
                    _stage(2 * K + 2, 0)
            _scatter_wait(b)
            if j + NBUF < SUPER:
                _fire_gather(j + NBUF, b)
            else:
                if j == SUPER - NBUF:
                    @pl.when(not_last)
                    def _w0():
                        _stage_wait(0)

                @pl.when(not_last)
                def _gnext():
                    _fire_gather(j + NBUF, b)

        @pl.when(not_last)
        def _prefetch():
            _stage(2 * K + 3, 1)

        return 0

    lax.fori_loop(0, NSUPER, _super, 0)

    plsc.subcore_barrier()

    obase = s * ZROWS
    pltpu.sync_copy(acc.at[pl.ds(obase, ZROWS)],
                    out_hbm.at[c, pl.ds(obase, ZROWS)])


def kernel(feat, edge_index, W, al, ar):
    del al, ar
    idx = _pad_edges(edge_index)
    h = _project(feat, W)
    partials = _sc_segment_sum(h, idx)
    return _combine(partials)


# speedup vs baseline: 45.7014x; 1.0199x over previous
import functools

import jax
import jax.numpy as jnp
from jax import lax
from jax.experimental import pallas as pl
from jax.experimental.pallas import tpu as pltpu
from jax.experimental.pallas import tpu_sc as plsc

N_NODES = 10000
D = 128
N_EDGES = 320000

NC = 2
NS = 16
NW = NC * NS
MICRO = 128
NBUF = 2
M_PER_TILE = 80
CH = 8
SUPER = 16
NSUPER = M_PER_TILE // SUPER
EP = NW * M_PER_TILE * MICRO
ROWS2D = EP // MICRO
OCTETS = ROWS2D // CH
OCT_PER_TILE = M_PER_TILE // CH
ACC_ROWS = 10240
ZROWS = ACC_ROWS // NS
E_ROWS = N_EDGES // MICRO
E_ALIGNED = 2496


def _pad_body(e_ref, o_ref):
    tail = ROWS2D - E_ROWS
    flat = (jax.lax.broadcasted_iota(jnp.int32, (tail, MICRO), 0) * MICRO
            + jax.lax.broadcasted_iota(jnp.int32, (tail, MICRO), 1))
    pad_src = flat % N_NODES
    pad_dst = N_NODES + jax.lax.rem(flat, ACC_ROWS - N_NODES)
    src = jnp.concatenate(
        [e_ref[0].reshape(E_ROWS, MICRO), pad_src], axis=0)
    dst = jnp.concatenate(
        [e_ref[1].reshape(E_ROWS, MICRO), pad_dst], axis=0)
    o_ref[:, 0] = src.reshape(OCTETS, CH, MICRO)
    o_ref[:, 1] = dst.reshape(OCTETS, CH, MICRO)


def _pad_edges(edge_index):
    return pl.pallas_call(
        _pad_body,
        in_specs=[pl.BlockSpec((2, N_EDGES), lambda: (0, 0))],
        out_specs=pl.BlockSpec((OCTETS, 2, CH, MICRO), lambda: (0, 0, 0, 0)),
        out_shape=jax.ShapeDtypeStruct((OCTETS, 2, CH, MICRO), jnp.int32),
    )(edge_index)


def _mm_body(x_ref, w_ref, o_ref):
    o_ref[...] = jnp.dot(x_ref[...], w_ref[...],
                         preferred_element_type=jnp.float32)


def _project(feat, W):
    return pl.pallas_call(
        _mm_body,
        grid=(5,),
        in_specs=[
            pl.BlockSpec((N_NODES // 5, D), lambda i: (i, 0)),
            pl.BlockSpec((D, D), lambda i: (0, 0)),
        ],
        out_specs=pl.BlockSpec((N_NODES // 5, D), lambda i: (i, 0)),
        out_shape=jax.ShapeDtypeStruct((N_NODES, D), jnp.float32),
    )(feat, W)


def _add_body(p_ref, o_ref):
    o_ref[...] = p_ref[0] + p_ref[1]


def _combine(partials):
    return pl.pallas_call(
        _add_body,
        grid=(5,),
        in_specs=[pl.BlockSpec((2, N_NODES // 5, D), lambda i: (0, i, 0))],
        out_specs=pl.BlockSpec((N_NODES // 5, D), lambda i: (i, 0)),
        out_shape=jax.ShapeDtypeStruct((N_NODES, D), jnp.float32),
    )(partials)


@functools.partial(
    pl.kernel,
    out_type=jax.ShapeDtypeStruct((NC, ACC_ROWS, D), jnp.float32),
    mesh=plsc.VectorSubcoreMesh(core_axis_name="c", subcore_axis_name="s"),
    scratch_types=[
        pltpu.VMEM_SHARED((ACC_ROWS, D), jnp.float32),
        pltpu.VMEM((2, 2, CH, MICRO), jnp.int32),
        pltpu.VMEM((NBUF, MICRO, D), jnp.float32),
        pltpu.SemaphoreType.DMA,
        pltpu.SemaphoreType.DMA,
        pltpu.SemaphoreType.DMA,
        pltpu.SemaphoreType.DMA,
        pltpu.SemaphoreType.DMA,
        pltpu.SemaphoreType.DMA,
    ],
)
def _sc_segment_sum(h_hbm, idx_hbm, out_hbm,
                    acc, idxv, bufs, g0, g1, c0, c1, i0, i1):
    gsems = (g0, g1)
    ssems = (c0, c1)
    isems = (i0, i1)
    c = lax.axis_index("c")
    s = lax.axis_index("s")
    wid = c * NS + s
    oct0 = wid * OCT_PER_TILE

    def _stage(k, p):
        pltpu.async_copy(idx_hbm.at[oct0 + k], idxv.at[p], isems[p])

    def _stage_wait(p):
        pltpu.make_async_copy(idx_hbm.at[0], idxv.at[p], isems[p]).wait()

    def _fire_gather(j, b):
        m = j % SUPER
        pltpu.async_copy(
            h_hbm.at[idxv.at[m // CH, 0, m % CH]], bufs.at[b], gsems[b])

    def _gather_wait(b):
        pltpu.make_async_copy(
            h_hbm.at[pl.ds(0, MICRO)], bufs.at[b], gsems[b]).wait()

    def _scatter_wait(b):
        pltpu.make_async_copy(
            bufs.at[b], acc.at[pl.ds(0, MICRO)], ssems[b]).wait()

    _stage(0, 0)
    _stage(1, 1)

    zbuf = bufs.at[0]
    zero16 = jnp.zeros((16,), jnp.float32)

    def _zero_row(i, _):
        for cc in range(D // 16):
            zbuf[i, pl.ds(cc * 16, 16)] = zero16
        return 0

    lax.fori_loop(0, MICRO, _zero_row, 0)
    zbase = s * ZROWS
    zd = [
        pltpu.async_copy(zbuf, acc.at[pl.ds(zbase + k * MICRO, MICRO)],
                         gsems[k % NBUF])
        for k in range(ZROWS // MICRO)
    ]
    for d in zd:
        d.wait()

    plsc.subcore_barrier()

    _stage_wait(0)
    _fire_gather(0, 0)
    _fire_gather(1, 1)

    def _super(K, _):
        not_last = K < NSUPER - 1
        for j in range(SUPER):
            b = j % NBUF
            _gather_wait(b)
            pltpu.async_copy(
                bufs.at[b], acc.at[idxv.at[j // CH, 1, j % CH]], ssems[b],
                add=True)
            if j == CH - NBUF:
                _stage_wait(1)
            if j == CH:
                @pl.when(not_last)
                def _pf0():
                    _stage(2 * K + 2, 0)
            _scatter_wait(b)
            if j + NBUF < SUPER:
                _fire_gather(j + NBUF, b)
            else:
                if j == SUPER - NBUF:
                    @pl.when(not_last)
                    def _w0():
                        _stage_wait(0)

                @pl.when(not_last)
                def _gnext():
                    _fire_gather(j + NBUF, b)

        @pl.when(not_last)
        def _prefetch():
            _stage(2 * K + 3, 1)

        return 0

    lax.fori_loop(0, NSUPER, _super, 0)

    plsc.subcore_barrier()

    obase = s * ZROWS
    pltpu.sync_copy(acc.at[pl.ds(obase, ZROWS)],
                    out_hbm.at[c, pl.ds(obase, ZROWS)])


def kernel(feat, edge_index, W, al, ar):
    del al, ar
    idx = _pad_edges(edge_index)
    h = _project(feat, W)
    partials = _sc_segment_sum(h, idx)
    return _combine(partials)
